# baseline (speedup 1.0000x reference)
import functools

import jax
import jax.numpy as jnp
from jax import lax
from jax.experimental import pallas as pl
from jax.experimental.pallas import tpu as pltpu
from jax.experimental.pallas import tpu_sc as plsc

_NC, _NS, _L = 2, 16, 16
_NW = _NC * _NS
_WCOL = 4
_WIDS = _WCOL * 128
_SROWS = 64

_CP = pltpu.CompilerParams(use_tc_tiling_on_sc=True, needs_layout_passes=False)


@functools.lru_cache(maxsize=None)
def _build(batch, emb_dim, vocab):
    ncols = (vocab + 127) // 128
    vocab_pad = ncols * 128
    wcols = (ncols + _NS - 1) // _NS
    bw = batch // _NW
    nscr = 2 * batch + 8
    dump = 2 * batch
    mesh = plsc.VectorSubcoreMesh(
        core_axis_name="c", subcore_axis_name="s",
        num_cores=_NC, num_subcores=_NS)

    @functools.partial(
        pl.kernel,
        out_type=jax.ShapeDtypeStruct((nscr, 128), jnp.float32),
        mesh=mesh,
        compiler_params=_CP,
        scratch_types=[
            pltpu.VMEM((2, 2048), jnp.int32),
            pltpu.VMEM((batch,), jnp.int32),
            pltpu.VMEM((batch,), jnp.int32),
            pltpu.VMEM((batch,), jnp.int32),
            pltpu.VMEM((batch,), jnp.int32),
            pltpu.VMEM((emb_dim, _WIDS), jnp.float32),
            pltpu.VMEM((_SROWS, 128), jnp.float32),
            pltpu.VMEM((_SROWS,), jnp.int32),
            pltpu.SemaphoreType.DMA,
            pltpu.SemaphoreType.DMA,
        ],
    )
    def gather_k(xT_hbm, uT_hbm, iT_hbm, scr_hbm,
                 idc, eid, ecode, bid, bcode, win, stage, sidx, wsem, ssem):
        wid = lax.axis_index("s") * _NC + lax.axis_index("c")
        s = wid // _NS
        half = wid % _NS
        lo = half * wcols * 128
        hi = jnp.minimum(lo + wcols * 128, vocab)
        iota = lax.iota(jnp.int32, _L)
        ones = iota >= 0
        dumpv = jnp.broadcast_to(jnp.int32(dump), (_L,))

        def filt_chunk(ch, cur):
            pltpu.sync_copy(xT_hbm.at[:, pl.ds(ch * 2048, 2048)], idc)

            def filt_vec(q, cur):
                v = idc[s, pl.ds(q * _L, _L)]
                slot = ch * 2048 + q * _L + iota
                m = (v >= lo) & (v < hi)
                plsc.store_compressed(eid.at[pl.ds(cur, _L)], v, mask=m)
                plsc.store_compressed(
                    ecode.at[pl.ds(cur, _L)], 2 * slot + s, mask=m)
                return cur + plsc.all_reduce_population_count(m)[0]

            return lax.fori_loop(0, 2048 // _L, filt_vec, cur)

        cur = lax.fori_loop(0, batch // 2048, filt_chunk, jnp.int32(0))
        nev = (cur + _L - 1) // _L

        def do_window(w, rs):
            wlo = lo + w * _WIDS
            wstart = jnp.minimum(wlo, jnp.int32(vocab_pad - _WIDS))

            def bin_vec(qb, nb):
                v = eid[pl.ds(qb * _L, _L)]
                c = ecode[pl.ds(qb * _L, _L)]
                m = (v >= wlo) & (v < wlo + _WIDS) & (qb * _L + iota < cur)
                plsc.store_compressed(bid.at[pl.ds(nb, _L)], v, mask=m)
                plsc.store_compressed(bcode.at[pl.ds(nb, _L)], c, mask=m)
                return nb + plsc.all_reduce_population_count(m)[0]

            nb = lax.fori_loop(0, nev, bin_vec, jnp.int32(0))
            plsc.store_compressed(
                bid.at[pl.ds(nb, _L)], jnp.broadcast_to(wstart, (_L,)),
                mask=ones)
            plsc.store_compressed(
                bcode.at[pl.ds(nb, _L)], dumpv, mask=ones)

            woff = pl.multiple_of(wstart, 128)

            @pl.when((nb > 0) & (s == 0))
            def _():
                pltpu.async_copy(
                    uT_hbm.at[:, pl.ds(woff, _WIDS)], win, wsem).wait()

            @pl.when((nb > 0) & (s == 1))
            def _():
                pltpu.async_copy(
                    iT_hbm.at[:, pl.ds(woff, _WIDS)], win, wsem).wait()

            def proc_block(tb, rs):
                bv = bid[pl.ds(tb * _L, _L)]
                cv = bcode[pl.ds(tb * _L, _L)]
                plsc.store_compressed(sidx.at[pl.ds(rs, _L)], cv, mask=ones)
                for k in range(_L):
                    lane = jnp.broadcast_to(bv[k] - wstart, (_L,))
                    v0 = plsc.load_gather(win, [iota, lane])
                    stage[rs + k, pl.ds(0, _L)] = v0
                    if emb_dim == 2 * _L:
                        v1 = plsc.load_gather(win, [iota + _L, lane])
                        stage[rs + k, pl.ds(_L, _L)] = v1
                rs = rs + _L

                @pl.when(rs >= _SROWS)
                def _():
                    pltpu.async_copy(stage, scr_hbm.at[sidx], ssem).wait()

                return jnp.where(rs >= _SROWS, jnp.int32(0), rs)

            nblk = (nb + _L - 1) // _L
            return lax.fori_loop(0, nblk, proc_block, rs)

        nwin_w = (jnp.minimum(lo + wcols * 128, jnp.int32(vocab_pad)) - lo
                  + _WIDS - 1) // _WIDS
        rs = lax.fori_loop(0, nwin_w, do_window, jnp.int32(0))

        @pl.when(rs > 0)
        def _():
            def pad_blk(pb, c):
                @pl.when(pb * _L >= rs)
                def _():
                    plsc.store_compressed(
                        sidx.at[pl.ds(pb * _L, _L)], dumpv, mask=ones)
                return c

            lax.fori_loop(0, _SROWS // _L, pad_blk, 0)
            pltpu.async_copy(stage, scr_hbm.at[sidx], ssem).wait()

    @functools.partial(
        pl.kernel,
        out_type=jax.ShapeDtypeStruct((2, emb_dim, batch), jnp.float32),
        mesh=mesh,
        compiler_params=_CP,
        scratch_types=[
            pltpu.VMEM((128, 128), jnp.float32),
            pltpu.VMEM((emb_dim, batch // _NW), jnp.float32),
            pltpu.VMEM((emb_dim, batch // _NW), jnp.float32),
            pltpu.SemaphoreType.DMA,
        ],
    )
    def unperm_k(scr_hbm, out_hbm, chunk, uslab, islab, osem):
        wid = lax.axis_index("s") * _NC + lax.axis_index("c")
        base = wid * bw
        iota = lax.iota(jnp.int32, _L)
        slabs = [uslab, islab]

        def do_chunk(ch, carry):
            pltpu.sync_copy(
                scr_hbm.at[pl.ds(2 * base + ch * 128, 128)], chunk)

            def do_block(b2, carry2):
                for k in range(_L):
                    r = b2 * _L + k
                    pos = jnp.broadcast_to(ch * 64 + b2 * 8 + k // 2, (_L,))
                    slab = slabs[k % 2]
                    v0 = chunk[r, pl.ds(0, _L)]
                    plsc.store_scatter(slab, [iota, pos], v0)
                    if emb_dim == 2 * _L:
                        v1 = chunk[r, pl.ds(_L, _L)]
                        plsc.store_scatter(slab, [iota + _L, pos], v1)
                return carry2

            return lax.fori_loop(0, 8, do_block, carry)

        lax.fori_loop(0, (2 * bw) // 128, do_chunk, 0)
        c0 = pltpu.async_copy(uslab, out_hbm.at[0].at[:, pl.ds(base, bw)],
                              osem)
        c1 = pltpu.async_copy(islab, out_hbm.at[1].at[:, pl.ds(base, bw)],
                              osem)
        c0.wait()
        c1.wait()

    def run(xT, uT, iT):
        scr = gather_k(xT, uT, iT)
        return unperm_k(scr)

    return run


def kernel(x, uid_table, iid_table):
    batch = x.shape[0]
    vocab, emb_dim = uid_table.shape
    out3 = _build(batch, emb_dim, vocab)(x.T, uid_table.T, iid_table.T)
    return lax.transpose(out3, (2, 0, 1))


# speedup vs baseline: 6.0569x; 6.0569x over previous
import functools

import jax
import jax.numpy as jnp
from jax import lax
from jax.experimental import pallas as pl
from jax.experimental.pallas import tpu as pltpu
from jax.experimental.pallas import tpu_sc as plsc

_NC, _NS, _L = 2, 16, 16
_NW = _NC * _NS
_NSLOT = 16


@functools.lru_cache(maxsize=None)
def _build(batch, emb_dim):
    bw = batch // _NW
    nblk = bw // _NSLOT
    mesh = plsc.VectorSubcoreMesh(
        core_axis_name="c", subcore_axis_name="s",
        num_cores=_NC, num_subcores=_NS)

    slot_types = [pltpu.VMEM((emb_dim, 128), jnp.float32)] * _NSLOT
    sem_types = [pltpu.SemaphoreType.DMA] * _NSLOT

    @functools.partial(
        pl.kernel,
        out_type=jax.ShapeDtypeStruct((2, emb_dim, batch), jnp.float32),
        mesh=mesh,
        compiler_params=pltpu.CompilerParams(
            use_tc_tiling_on_sc=True, needs_layout_passes=False),
        scratch_types=[
            pltpu.VMEM((2, bw), jnp.int32),
            pltpu.VMEM((emb_dim, bw), jnp.float32),
            pltpu.VMEM((emb_dim, bw), jnp.float32),
            pltpu.SemaphoreType.DMA,
        ] + slot_types + sem_types,
    )
    def lookup(xT_hbm, uT_hbm, iT_hbm, out_hbm, idv, uslab, islab, osem,
               *slots_and_sems):
        slots = slots_and_sems[:_NSLOT]
        sems = slots_and_sems[_NSLOT:]
        wid = lax.axis_index("s") * _NC + lax.axis_index("c")
        base = wid * bw
        pltpu.sync_copy(xT_hbm.at[:, pl.ds(base, bw)], idv)

        iota = lax.iota(jnp.int32, _L)
        iota_hi = iota + _L if emb_dim == 2 * _L else None

        def run_table(t_hbm, slab, s):
            def window_starts(b):
                return idv[s, pl.ds(b * _NSLOT, _NSLOT)] & jnp.int32(~127)

            def fetch_block(b):
                jv = window_starts(b)
                for k in range(_NSLOT):
                    pltpu.async_copy(
                        t_hbm.at[:, pl.ds(pl.multiple_of(jv[k], 128), 128)],
                        slots[k], sems[k])

            fetch_block(0)

            def body(b, carry):
                lv = idv[s, pl.ds(b * _NSLOT, _NSLOT)] - window_starts(b)
                for k in range(_NSLOT):
                    pltpu.make_async_copy(
                        t_hbm.at[:, pl.ds(0, 128)], slots[k], sems[k]).wait()
                    lane = jnp.broadcast_to(lv[k], (_L,))
                    pos = jnp.broadcast_to(b * _NSLOT + k, (_L,))
                    v0 = plsc.load_gather(slots[k], [iota, lane])
                    plsc.store_scatter(slab, [iota, pos], v0)
                    if iota_hi is not None:
                        v1 = plsc.load_gather(slots[k], [iota_hi, lane])
                        plsc.store_scatter(slab, [iota_hi, pos], v1)

                @pl.when(b + 1 < nblk)
                def _():
                    fetch_block(b + 1)

                return carry

            lax.fori_loop(0, nblk, body, 0)

        run_table(uT_hbm, uslab, 0)
        plsc.subcore_barrier()
        ocp0 = pltpu.async_copy(
            uslab, out_hbm.at[0].at[:, pl.ds(base, bw)], osem)
        run_table(iT_hbm, islab, 1)
        plsc.subcore_barrier()
        ocp1 = pltpu.async_copy(
            islab, out_hbm.at[1].at[:, pl.ds(base, bw)], osem)
        ocp0.wait()
        ocp1.wait()

    return lookup


def kernel(x, uid_table, iid_table):
    batch = x.shape[0]
    emb_dim = uid_table.shape[1]
    out3 = _build(batch, emb_dim)(x.T, uid_table.T, iid_table.T)
    return lax.transpose(out3, (2, 0, 1))
